# baseline (speedup 1.0000x reference)
import functools

import jax
import jax.numpy as jnp
from jax import lax
from jax.experimental import pallas as pl
from jax.experimental.pallas import tpu as pltpu
from jax.experimental.pallas import tpu_sc as plsc

D = 128
L = 16
NC = 2
NS = 16
NW = NC * NS
C = 80
NB = 5


@functools.partial(jax.jit, static_argnames=("E", "N"))
def _score(h, edge_index, *, E, N):
    EW = E // NW
    NCH = EW // C

    mesh = plsc.VectorSubcoreMesh(
        core_axis_name="c", subcore_axis_name="s", num_cores=NC,
        num_subcores=NS)

    @functools.partial(
        pl.kernel,
        out_type=jax.ShapeDtypeStruct((E,), jnp.float32),
        mesh=mesh,
        scratch_types=[
            pltpu.VMEM((EW,), jnp.int32),
            pltpu.VMEM((EW,), jnp.int32),
            pltpu.VMEM((NB, C, D // 2), jnp.int32),
            pltpu.VMEM((NB, C, D // 2), jnp.int32),
            pltpu.VMEM((EW,), jnp.float32),
            pltpu.VMEM_SHARED((10000, D // 2), jnp.int32),
            *([pltpu.SemaphoreType.DMA] * 14),
        ],
        compiler_params=pltpu.CompilerParams(needs_layout_passes=False, use_tc_tiling_on_sc=False),
    )
    def k(h_hbm, edge_hbm, out_hbm,
          sidx, didx, srows, drows, scores, table,
          *sems):
        wid = lax.axis_index("s") * NC + lax.axis_index("c")
        base_w = wid * EW
        lane = lax.iota(jnp.int32, L)
        last_lane = lane == (L - 1)
        sems_s = sems[:NB]
        sems_d = sems[NB:2 * NB]

        cps = pltpu.async_copy(edge_hbm.at[0, pl.ds(base_w, EW)], sidx,
                               sems[2 * NB])
        cpd = pltpu.async_copy(edge_hbm.at[1, pl.ds(base_w, EW)], didx,
                               sems[2 * NB + 1])
        sid = lax.axis_index("s")
        rpt = 10000 // NS
        pltpu.sync_copy(h_hbm.at[pl.ds(sid * rpt, rpt)],
                        table.at[pl.ds(sid * rpt, rpt)])
        cps.wait()
        cpd.wait()
        plsc.subcore_barrier()

        def issue(chunk, b):
            pltpu.async_copy(
                table.at[sidx.at[pl.ds(chunk * C, C)]], srows.at[b],
                sems_s[b])
            pltpu.async_copy(
                h_hbm.at[didx.at[pl.ds(chunk * C, C)]], drows.at[b],
                sems_d[b])

        def wait(b):
            pltpu.make_async_copy(
                table.at[sidx.at[pl.ds(0, C)]], srows.at[b],
                sems_s[b]).wait()
            pltpu.make_async_copy(
                h_hbm.at[didx.at[pl.ds(0, C)]], drows.at[b],
                sems_d[b]).wait()

        def compute(chunk, b):
            wait(b)
            sr = srows.at[b]
            dr = drows.at[b]
            ebase = chunk * C

            @plsc.parallel_loop(0, C, unroll=4)
            def edge_body(e):
                acc_a = jnp.zeros((L,), jnp.float32)
                acc_b = jnp.zeros((L,), jnp.float32)
                for j in range(D // (2 * L)):
                    s2 = plsc.bitcast(sr[e, pl.ds(j * L, L)], jnp.bfloat16)
                    d2 = plsc.bitcast(dr[e, pl.ds(j * L, L)], jnp.bfloat16)
                    pa, pb = plsc.unpack(
                        s2 * d2, format=plsc.PackFormat.INTERLEAVED,
                        preferred_element_type=jnp.float32)
                    acc_a = acc_a + pa
                    acc_b = acc_b + pb
                csum = lax.cumsum(acc_a + acc_b)
                plsc.store_scatter(scores, [jnp.broadcast_to(ebase + e, (L,))],
                                   csum, mask=last_lane)

        for b in range(NB):
            issue(b, b)

        NMAIN = NCH - NCH % NB

        @pl.loop(0, NMAIN, step=NB)
        def chunk_loop(i):
            for b in range(NB):
                chunk = i + b
                compute(chunk, b)

                @pl.when(chunk + NB < NCH)
                def _():
                    issue(chunk + NB, b)

        for t in range(NCH % NB):
            compute(NMAIN + t, t)

        pltpu.sync_copy(scores, out_hbm.at[pl.ds(base_w, EW)])

    return k(h, edge_index)


def kernel(h, edge_index):
    N, d = h.shape
    E = edge_index.shape[1]
    hu = lax.bitcast_convert_type(h, jnp.uint32) + jnp.uint32(0x8000)
    hi = lax.bitcast_convert_type(
        (hu[:, : d // 2] >> 16) | (hu[:, d // 2:] & jnp.uint32(0xFFFF0000)),
        jnp.int32)
    out = _score(hi, edge_index.astype(jnp.int32), E=E, N=N)
    return out.reshape(E, 1)


# speedup vs baseline: 1.0023x; 1.0023x over previous
import functools

import jax
import jax.numpy as jnp
from jax import lax
from jax.experimental import pallas as pl
from jax.experimental.pallas import tpu as pltpu
from jax.experimental.pallas import tpu_sc as plsc

D = 128
L = 16
NC = 2
NS = 16
NW = NC * NS
C = 80
NB = 4


@functools.partial(jax.jit, static_argnames=("E", "N"))
def _score(h, edge_index, *, E, N):
    EW = E // NW
    NCH = EW // C

    mesh = plsc.VectorSubcoreMesh(
        core_axis_name="c", subcore_axis_name="s", num_cores=NC,
        num_subcores=NS)

    @functools.partial(
        pl.kernel,
        out_type=jax.ShapeDtypeStruct((E,), jnp.float32),
        mesh=mesh,
        scratch_types=[
            pltpu.VMEM((EW,), jnp.int32),
            pltpu.VMEM((EW,), jnp.int32),
            pltpu.VMEM((NB, C, D // 2), jnp.int32),
            pltpu.VMEM((NB, C, D // 2), jnp.int32),
            pltpu.VMEM((EW,), jnp.float32),
            pltpu.VMEM_SHARED((10000, D // 2), jnp.int32),
            *([pltpu.SemaphoreType.DMA] * 10),
        ],
        compiler_params=pltpu.CompilerParams(needs_layout_passes=False, use_tc_tiling_on_sc=False),
    )
    def k(h_hbm, edge_hbm, out_hbm,
          sidx, didx, srows, drows, scores, table,
          *sems):
        wid = lax.axis_index("s") * NC + lax.axis_index("c")
        base_w = wid * EW
        lane = lax.iota(jnp.int32, L)
        last_lane = lane == (L - 1)
        sems_s = sems[:NB]
        sems_d = sems[NB:2 * NB]

        cps = pltpu.async_copy(edge_hbm.at[0, pl.ds(base_w, EW)], sidx,
                               sems[2 * NB])
        cpd = pltpu.async_copy(edge_hbm.at[1, pl.ds(base_w, EW)], didx,
                               sems[2 * NB + 1])
        sid = lax.axis_index("s")
        rpt = 10000 // NS
        pltpu.sync_copy(h_hbm.at[pl.ds(sid * rpt, rpt)],
                        table.at[pl.ds(sid * rpt, rpt)])
        cps.wait()
        cpd.wait()
        plsc.subcore_barrier()

        def issue(chunk, b):
            pltpu.async_copy(
                table.at[sidx.at[pl.ds(chunk * C, C)]], srows.at[b],
                sems_s[b])
            pltpu.async_copy(
                h_hbm.at[didx.at[pl.ds(chunk * C, C)]], drows.at[b],
                sems_d[b])

        def wait(b):
            pltpu.make_async_copy(
                table.at[sidx.at[pl.ds(0, C)]], srows.at[b],
                sems_s[b]).wait()
            pltpu.make_async_copy(
                h_hbm.at[didx.at[pl.ds(0, C)]], drows.at[b],
                sems_d[b]).wait()

        def compute(chunk, b):
            wait(b)
            sr = srows.at[b]
            dr = drows.at[b]
            ebase = chunk * C

            @plsc.parallel_loop(0, C, unroll=4)
            def edge_body(e):
                acc_a = jnp.zeros((L,), jnp.float32)
                acc_b = jnp.zeros((L,), jnp.float32)
                for j in range(D // (2 * L)):
                    s2 = plsc.bitcast(sr[e, pl.ds(j * L, L)], jnp.bfloat16)
                    d2 = plsc.bitcast(dr[e, pl.ds(j * L, L)], jnp.bfloat16)
                    pa, pb = plsc.unpack(
                        s2 * d2, format=plsc.PackFormat.INTERLEAVED,
                        preferred_element_type=jnp.float32)
                    acc_a = acc_a + pa
                    acc_b = acc_b + pb
                csum = lax.cumsum(acc_a + acc_b)
                plsc.store_scatter(scores, [jnp.broadcast_to(ebase + e, (L,))],
                                   csum, mask=last_lane)

        for b in range(NB):
            issue(b, b)

        NMAIN = NCH - NCH % NB

        @pl.loop(0, NMAIN, step=NB)
        def chunk_loop(i):
            for b in range(NB):
                chunk = i + b
                compute(chunk, b)

                @pl.when(chunk + NB < NCH)
                def _():
                    issue(chunk + NB, b)

        for t in range(NCH % NB):
            compute(NMAIN + t, t)

        pltpu.sync_copy(scores, out_hbm.at[pl.ds(base_w, EW)])

    return k(h, edge_index)


def kernel(h, edge_index):
    N, d = h.shape
    E = edge_index.shape[1]
    hu = lax.bitcast_convert_type(h, jnp.uint32) + jnp.uint32(0x8000)
    hi = lax.bitcast_convert_type(
        (hu[:, : d // 2] >> 16) | (hu[:, d // 2:] & jnp.uint32(0xFFFF0000)),
        jnp.int32)
    out = _score(hi, edge_index.astype(jnp.int32), E=E, N=N)
    return out.reshape(E, 1)
